# initial kernel scaffold (unmeasured)
import jax
import jax.numpy as jnp
from jax import lax
from jax.experimental import pallas as pl
from jax.experimental.pallas import tpu as pltpu


def kernel(
    x,
):
    def body(*refs):
        pass

    out_shape = jax.ShapeDtypeStruct(..., jnp.float32)
    return pl.pallas_call(body, out_shape=out_shape)(...)


# baseline (device time: 27995 ns/iter reference)
import jax
import jax.numpy as jnp
from jax import lax
from jax.experimental import pallas as pl
from jax.experimental.pallas import tpu as pltpu

N_DEV = 16


def kernel(x):
    m, n = x.shape

    def body(x_ref, out_ref, msg_ref, recv_ref, send_sem, recv_sem):
        my = lax.axis_index("i")

        v = x_ref[...]
        s = 1
        while s < m:
            shifted = jnp.concatenate(
                [jnp.ones((s, n), v.dtype), v[: m - s, :]], axis=0
            )
            v = v * shifted
            s *= 2

        @pl.when(my == 0)
        def _():
            recv_ref[...] = jnp.ones_like(recv_ref)

        @pl.when(my > 0)
        def _():
            rdma = pltpu.make_async_remote_copy(
                src_ref=msg_ref,
                dst_ref=recv_ref,
                send_sem=send_sem,
                recv_sem=recv_sem,
                device_id=((my - 1) % N_DEV,),
                device_id_type=pl.DeviceIdType.MESH,
            )
            rdma.wait_recv()

        prefix = recv_ref[...]

        @pl.when(my < N_DEV - 1)
        def _():
            msg_ref[...] = prefix * v[m - 1 :, :]
            rdma = pltpu.make_async_remote_copy(
                src_ref=msg_ref,
                dst_ref=recv_ref,
                send_sem=send_sem,
                recv_sem=recv_sem,
                device_id=((my + 1) % N_DEV,),
                device_id_type=pl.DeviceIdType.MESH,
            )
            rdma.start()
            rdma.wait_send()

        out_ref[...] = v * prefix

    return pl.pallas_call(
        body,
        out_shape=jax.ShapeDtypeStruct((m, n), jnp.float32),
        in_specs=[pl.BlockSpec(memory_space=pltpu.VMEM)],
        out_specs=pl.BlockSpec(memory_space=pltpu.VMEM),
        scratch_shapes=[
            pltpu.VMEM((1, n), jnp.float32),
            pltpu.VMEM((1, n), jnp.float32),
            pltpu.SemaphoreType.DMA,
            pltpu.SemaphoreType.DMA,
        ],
        compiler_params=pltpu.CompilerParams(has_side_effects=True),
    )(x)


# device time: 25115 ns/iter; 1.1147x vs baseline; 1.1147x over previous
import jax
import jax.numpy as jnp
from jax import lax
from jax.experimental import pallas as pl
from jax.experimental.pallas import tpu as pltpu

N_DEV = 16
ROUNDS = (1, 2, 4, 8)


def kernel(x):
    m, n = x.shape

    shifts = []
    s = 1
    while s < m:
        shifts.append(s)
        s *= 2
    n_rounds = len(ROUNDS)
    per_round = len(shifts) // n_rounds
    chunks = [
        shifts[r * per_round : (r + 1) * per_round] for r in range(n_rounds)
    ]
    head = shifts[n_rounds * per_round :]

    def hs_step(v, shift):
        shifted = jnp.concatenate(
            [jnp.ones((shift, n), v.dtype), v[: m - shift, :]], axis=0
        )
        return v * shifted

    def body(x_ref, out_ref, msg_ref, recv_ref, send_sems, recv_sems):
        my = lax.axis_index("i")

        xv = x_ref[...]
        t = xv
        rows = m
        while rows > 1:
            half = rows // 2
            t = t[:half, :] * t[half : 2 * half, :]
            rows = half

        e_val = jnp.ones((1, n), jnp.float32)
        s_val = t

        v = xv
        for sh in head:
            v = hs_step(v, sh)

        for r, d in enumerate(ROUNDS):
            @pl.when(my + d < N_DEV)
            def _():
                msg_ref[r, :, :] = s_val
                rdma = pltpu.make_async_remote_copy(
                    src_ref=msg_ref.at[r],
                    dst_ref=recv_ref.at[r],
                    send_sem=send_sems.at[r],
                    recv_sem=recv_sems.at[r],
                    device_id=((my + d) % N_DEV,),
                    device_id_type=pl.DeviceIdType.MESH,
                )
                rdma.start()

            for sh in chunks[r]:
                v = hs_step(v, sh)

            @pl.when(my >= d)
            def _():
                rdma = pltpu.make_async_remote_copy(
                    src_ref=msg_ref.at[r],
                    dst_ref=recv_ref.at[r],
                    send_sem=send_sems.at[r],
                    recv_sem=recv_sems.at[r],
                    device_id=((my - d) % N_DEV,),
                    device_id_type=pl.DeviceIdType.MESH,
                )
                rdma.wait_recv()

            @pl.when(my + d < N_DEV)
            def _():
                rdma = pltpu.make_async_remote_copy(
                    src_ref=msg_ref.at[r],
                    dst_ref=recv_ref.at[r],
                    send_sem=send_sems.at[r],
                    recv_sem=recv_sems.at[r],
                    device_id=((my + d) % N_DEV,),
                    device_id_type=pl.DeviceIdType.MESH,
                )
                rdma.wait_send()

            q = recv_ref[r, :, :]
            valid = my >= d
            e_val = jnp.where(valid, q * e_val, e_val)
            s_val = jnp.where(valid, q * s_val, s_val)

        out_ref[...] = v * e_val

    return pl.pallas_call(
        body,
        out_shape=jax.ShapeDtypeStruct((m, n), jnp.float32),
        in_specs=[pl.BlockSpec(memory_space=pltpu.VMEM)],
        out_specs=pl.BlockSpec(memory_space=pltpu.VMEM),
        scratch_shapes=[
            pltpu.VMEM((len(ROUNDS), 1, n), jnp.float32),
            pltpu.VMEM((len(ROUNDS), 1, n), jnp.float32),
            pltpu.SemaphoreType.DMA((len(ROUNDS),)),
            pltpu.SemaphoreType.DMA((len(ROUNDS),)),
        ],
        compiler_params=pltpu.CompilerParams(has_side_effects=True),
    )(x)
